# baseline (speedup 1.0000x reference)
import functools
import jax
import jax.numpy as jnp
from jax import lax
from jax.experimental import pallas as pl
from jax.experimental.pallas import tpu as pltpu
from jax.experimental.pallas import tpu_sc as plsc

N = 10000
E = 320000
D = 128
DH = D // 2

NC, NS = 2, 16
NW = NC * NS
EPW = E // NW
EPT = E // NS
CH = 125
ACHUNK = EPT // CH
ABUF = 4
NPAD = 10240

PPW = 2 * E // NW
PPAD = 20480
SCH = 80
SCHUNK = PPAD // SCH
SBUF = 2

_mesh = plsc.VectorSubcoreMesh(core_axis_name="c", subcore_axis_name="s")


@functools.partial(
    pl.kernel,
    out_type=(
        jax.ShapeDtypeStruct((NW, NPAD), jnp.float32),
        jax.ShapeDtypeStruct((NW, NPAD), jnp.float32),
    ),
    mesh=_mesh,
    scratch_types=[
        pltpu.VMEM((EPW,), jnp.int32),
        pltpu.VMEM((NPAD,), jnp.float32),
        pltpu.VMEM((NPAD,), jnp.float32),
    ],
    compiler_params=pltpu.CompilerParams(needs_layout_passes=False),
)
def _deg_kernel(src_hbm, dst_hbm, dout_hbm, din_hbm, idx_v, hist0, hist1):
    w = lax.axis_index("s") * NC + lax.axis_index("c")
    zeros = jnp.zeros((16,), jnp.float32)
    ones = jnp.ones((16,), jnp.float32)

    def zero_body(i, _):
        hist0[pl.ds(i * 16, 16)] = zeros
        hist1[pl.ds(i * 16, 16)] = zeros
        return 0

    lax.fori_loop(0, NPAD // 16, zero_body, 0)

    for ref, hist, out in ((src_hbm, hist0, dout_hbm), (dst_hbm, hist1, din_hbm)):
        pltpu.sync_copy(ref.at[w], idx_v)

        def acc_body(i, _, hist=hist):
            idx = idx_v[pl.ds(i * 16, 16)]
            plsc.addupdate_scatter(hist, [idx], ones)
            return 0

        lax.fori_loop(0, EPW // 16, acc_body, 0)
        pltpu.sync_copy(hist, out.at[w])


@functools.partial(
    pl.kernel,
    out_type=jax.ShapeDtypeStruct((NC, NPAD, DH), jnp.float32),
    mesh=_mesh,
    scratch_types=[
        pltpu.VMEM((ACHUNK, CH), jnp.int32),
        pltpu.VMEM((ACHUNK, CH), jnp.int32),
        pltpu.VMEM((ABUF, CH), jnp.int32),
        pltpu.VMEM((ABUF, CH, DH), jnp.float32),
        pltpu.VMEM_SHARED((NPAD, DH), jnp.float32),
        pltpu.SemaphoreType.DMA((ABUF,)),
    ],
    compiler_params=pltpu.CompilerParams(use_tc_tiling_on_sc=False),
)
def _agg_kernel(src_hbm, dst_hbm, y0_hbm, y1_hbm, out_hbm,
                sidx, didx, didx_s, rows, agg_sh, gsem):
    c = lax.axis_index("c")
    s = lax.axis_index("s")

    pltpu.sync_copy(src_hbm.at[s], sidx)
    pltpu.sync_copy(dst_hbm.at[s], didx)

    def stage_didx(b, j):
        for k in range(7):
            didx_s[b, pl.ds(k * 16, 16)] = didx[j, pl.ds(k * 16, 16)]
        didx_s[b, pl.ds(CH - 16, 16)] = didx[j, pl.ds(CH - 16, 16)]

    zeros = jnp.zeros((16,), jnp.float32)

    def zrow(i, _):
        for k in range(DH // 16):
            rows[0, i, pl.ds(k * 16, 16)] = zeros
        return 0

    lax.fori_loop(0, CH, zrow, 0)
    for k in range(8):
        pltpu.sync_copy(rows.at[0, pl.ds(0, 80)],
                        agg_sh.at[pl.ds(s * 640 + k * 80, 80)])
    plsc.subcore_barrier()

    def gather(j, b):
        @pl.when(c == 0)
        def _():
            pltpu.async_copy(y0_hbm.at[sidx.at[j]], rows.at[b], gsem.at[b])

        @pl.when(c == 1)
        def _():
            pltpu.async_copy(y1_hbm.at[sidx.at[j]], rows.at[b], gsem.at[b])

    def gwait(j, b):
        @pl.when(c == 0)
        def _():
            pltpu.make_async_copy(y0_hbm.at[sidx.at[j]], rows.at[b],
                                  gsem.at[b]).wait()

        @pl.when(c == 1)
        def _():
            pltpu.make_async_copy(y1_hbm.at[sidx.at[j]], rows.at[b],
                                  gsem.at[b]).wait()

    for b in range(ABUF):
        stage_didx(b, b)
        gather(b, b)

    def outer(jo, _):
        for b in range(ABUF):
            j = jo * ABUF + b
            gwait(j, b)
            pltpu.sync_copy(rows.at[b], agg_sh.at[didx_s.at[b]], add=True)

            @pl.when(jo < ACHUNK // ABUF - 1)
            def _():
                jn = j + ABUF
                stage_didx(b, jn)
                gather(jn, b)
        return 0

    lax.fori_loop(0, ACHUNK // ABUF, outer, 0)
    plsc.subcore_barrier()

    for k in range(8):
        sl = pl.ds(s * 640 + k * 80, 80)
        pltpu.sync_copy(agg_sh.at[sl], out_hbm.at[c, sl])


@functools.partial(
    pl.kernel,
    out_type=jax.ShapeDtypeStruct((NW, PPAD), jnp.float32),
    mesh=_mesh,
    scratch_types=[
        pltpu.VMEM((PPAD,), jnp.int32),
        pltpu.VMEM((PPAD,), jnp.int32),
        pltpu.VMEM((SBUF, SCH, D), jnp.float32),
        pltpu.VMEM((SBUF, SCH, D), jnp.float32),
        pltpu.VMEM((PPAD,), jnp.float32),
        pltpu.SemaphoreType.DMA((SBUF,)),
        pltpu.SemaphoreType.DMA((SBUF,)),
    ],
    compiler_params=pltpu.CompilerParams(needs_layout_passes=False),
)
def _score_kernel(pidx_hbm, qidx_hbm, h_hbm, out_hbm,
                  sidx, didx, hs, hd, sco, asem, bsem):
    w = lax.axis_index("s") * NC + lax.axis_index("c")
    pltpu.sync_copy(pidx_hbm.at[w], sidx)
    pltpu.sync_copy(qidx_hbm.at[w], didx)

    def gstart(j, b):
        pltpu.async_copy(h_hbm.at[sidx.at[pl.ds(j * SCH, SCH)]], hs.at[b],
                         asem.at[b])
        pltpu.async_copy(h_hbm.at[didx.at[pl.ds(j * SCH, SCH)]], hd.at[b],
                         bsem.at[b])

    def gwait(j, b):
        pltpu.make_async_copy(h_hbm.at[sidx.at[pl.ds(j * SCH, SCH)]],
                              hs.at[b], asem.at[b]).wait()
        pltpu.make_async_copy(h_hbm.at[didx.at[pl.ds(j * SCH, SCH)]],
                              hd.at[b], bsem.at[b]).wait()

    for b in range(SBUF):
        gstart(b, b)

    def outer(jo, _):
        for b in range(SBUF):
            j = jo * SBUF + b
            gwait(j, b)

            lanes = lax.iota(jnp.int32, 16)

            def blk16(g, _, b=b, j=j):
                sums = jnp.zeros((16,), jnp.float32)
                for u in range(16):
                    e = g * 16 + u
                    acc = hs[b, e, pl.ds(0, 16)] * hd[b, e, pl.ds(0, 16)]
                    for k in range(1, D // 16):
                        acc = acc + (hs[b, e, pl.ds(k * 16, 16)]
                                     * hd[b, e, pl.ds(k * 16, 16)])
                    sums = jnp.where(lanes == u, jnp.sum(acc), sums)
                sco[pl.ds(j * SCH + g * 16, 16)] = sums
                return 0

            lax.fori_loop(0, SCH // 16, blk16, 0)

            @pl.when(jo < SCHUNK // SBUF - 1)
            def _():
                gstart(j + SBUF, b)
        return 0

    lax.fori_loop(0, SCHUNK // SBUF, outer, 0)
    pltpu.sync_copy(sco, out_hbm.at[w])


def _norm_body(dout_ref, din_ref, ns_ref, nd_ref):
    dout = jnp.sum(dout_ref[...], axis=0, keepdims=True)
    din = jnp.sum(din_ref[...], axis=0, keepdims=True)
    ns_ref[...] = jnp.where(dout > 0.0, lax.rsqrt(jnp.maximum(dout, 1.0)), 0.0)
    nd_ref[...] = jnp.where(din > 0.0, lax.rsqrt(jnp.maximum(din, 1.0)), 0.0)


def _norm_kernel(doutp, dinp):
    return pl.pallas_call(
        _norm_body,
        out_shape=(
            jax.ShapeDtypeStruct((1, NPAD), jnp.float32),
            jax.ShapeDtypeStruct((1, NPAD), jnp.float32),
        ),
    )(doutp, dinp)


BM = 2000


def _mm1_body(x_ref, ns_ref, w_ref, olo_ref, ohi_ref):
    xs = x_ref[...] * ns_ref[...]
    y = jnp.dot(xs, w_ref[...], preferred_element_type=jnp.float32)
    olo_ref[...] = y[:, :DH]
    ohi_ref[...] = y[:, DH:]


def _mm1_kernel(x, ns, w):
    return pl.pallas_call(
        _mm1_body,
        grid=(N // BM,),
        in_specs=[
            pl.BlockSpec((BM, D), lambda i: (i, 0)),
            pl.BlockSpec((BM, 1), lambda i: (i, 0)),
            pl.BlockSpec((D, D), lambda i: (0, 0)),
        ],
        out_specs=[pl.BlockSpec((BM, DH), lambda i: (i, 0))] * 2,
        out_shape=[jax.ShapeDtypeStruct((N, DH), jnp.float32)] * 2,
    )(x, ns, w)


def _mid_body(alo_ref, ahi_ref, nd_ref, b1_ref, ns_ref, w_ref,
              olo_ref, ohi_ref):
    nd = nd_ref[...]
    ns = ns_ref[...]
    hlo = alo_ref[0] * nd + b1_ref[:, :DH]
    hhi = ahi_ref[0] * nd + b1_ref[:, DH:]
    hlo = jnp.where(hlo > 0.0, hlo, 0.01 * hlo) * ns
    hhi = jnp.where(hhi > 0.0, hhi, 0.01 * hhi) * ns
    y = (
        jnp.dot(hlo, w_ref[:DH, :], preferred_element_type=jnp.float32)
        + jnp.dot(hhi, w_ref[DH:, :], preferred_element_type=jnp.float32)
    )
    olo_ref[...] = y[:, :DH]
    ohi_ref[...] = y[:, DH:]


def _mid_kernel(aggp, nd, b1r, ns, w):
    return pl.pallas_call(
        _mid_body,
        grid=(N // BM,),
        in_specs=[
            pl.BlockSpec((1, BM, DH), lambda i: (0, i, 0)),
            pl.BlockSpec((1, BM, DH), lambda i: (1, i, 0)),
            pl.BlockSpec((BM, 1), lambda i: (i, 0)),
            pl.BlockSpec((1, D), lambda i: (0, 0)),
            pl.BlockSpec((BM, 1), lambda i: (i, 0)),
            pl.BlockSpec((D, D), lambda i: (0, 0)),
        ],
        out_specs=[pl.BlockSpec((BM, DH), lambda i: (i, 0))] * 2,
        out_shape=[jax.ShapeDtypeStruct((N, DH), jnp.float32)] * 2,
    )(aggp, aggp, nd, b1r, ns, w)


def _fin_body(alo_ref, ahi_ref, nd_ref, b2_ref, o_ref):
    nd = nd_ref[...]
    hlo = alo_ref[0] * nd + b2_ref[:, :DH]
    hhi = ahi_ref[0] * nd + b2_ref[:, DH:]
    o_ref[...] = jnp.concatenate([hlo, hhi], axis=-1)


def _fin_kernel(aggp, nd, b2r):
    return pl.pallas_call(
        _fin_body,
        grid=(N // BM,),
        in_specs=[
            pl.BlockSpec((1, BM, DH), lambda i: (0, i, 0)),
            pl.BlockSpec((1, BM, DH), lambda i: (1, i, 0)),
            pl.BlockSpec((BM, 1), lambda i: (i, 0)),
            pl.BlockSpec((1, D), lambda i: (0, 0)),
        ],
        out_specs=pl.BlockSpec((BM, D), lambda i: (i, 0)),
        out_shape=jax.ShapeDtypeStruct((N, D), jnp.float32),
    )(aggp, aggp, nd, b2r)


BD = 2000


def _dot_body(a_ref, b_ref, c_ref, d_ref, p_ref, n_ref):
    a = a_ref[...].astype(jnp.float32)
    b = b_ref[...].astype(jnp.float32)
    c = c_ref[...].astype(jnp.float32)
    d = d_ref[...].astype(jnp.float32)
    p_ref[...] = jnp.sum(a * b, axis=-1, keepdims=True)
    n_ref[...] = jnp.sum(c * d, axis=-1, keepdims=True)


_NBLK = E // BD


def _dot_kernel(g):
    return pl.pallas_call(
        _dot_body,
        grid=(_NBLK,),
        in_specs=[
            pl.BlockSpec((BD, D), lambda i: (i, 0)),
            pl.BlockSpec((BD, D), lambda i: (i + _NBLK, 0)),
            pl.BlockSpec((BD, D), lambda i: (i + 2 * _NBLK, 0)),
            pl.BlockSpec((BD, D), lambda i: (i + 3 * _NBLK, 0)),
        ],
        out_specs=[pl.BlockSpec((BD, 1), lambda i: (i, 0))] * 2,
        out_shape=[jax.ShapeDtypeStruct((E, 1), jnp.float32)] * 2,
    )(g, g, g, g)


@jax.jit
def kernel(edge_index, neg_edge_index, embed, W1, b1, W2, b2):
    src = edge_index[0]
    dst = edge_index[1]

    srcw = src.reshape(NW, EPW)
    dstw = dst.reshape(NW, EPW)
    srct = src.reshape(NS, ACHUNK, CH)
    dstt = dst.reshape(NS, ACHUNK, CH)

    doutp, dinp = _deg_kernel(srcw, dstw)
    nso, ndo = _norm_kernel(doutp, dinp)
    ns = nso[0, :N].reshape(N, 1)
    nd = ndo[0, :N].reshape(N, 1)

    y1lo, y1hi = _mm1_kernel(embed, ns, W1)
    aggp1 = _agg_kernel(srct, dstt, y1lo, y1hi)
    y2lo, y2hi = _mid_kernel(aggp1, nd, b1.reshape(1, D), ns, W2)
    aggp2 = _agg_kernel(srct, dstt, y2lo, y2hi)
    h = _fin_kernel(aggp2, nd, b2.reshape(1, D))

    pad = ((0, 0), (0, PPAD - PPW))
    pidx = jnp.pad(jnp.concatenate([src, neg_edge_index[0]]).reshape(NW, PPW),
                   pad)
    qidx = jnp.pad(jnp.concatenate([dst, neg_edge_index[1]]).reshape(NW, PPW),
                   pad)
    scores = _score_kernel(pidx, qidx, h)
    sflat = scores[:, :PPW].reshape(2 * E)
    pos = sflat[:E].reshape(E, 1)
    neg = sflat[E:].reshape(E, 1)
    return pos, neg


# speedup vs baseline: 1.7275x; 1.7275x over previous
import functools
import jax
import jax.numpy as jnp
from jax import lax
from jax.experimental import pallas as pl
from jax.experimental.pallas import tpu as pltpu
from jax.experimental.pallas import tpu_sc as plsc

N = 10000
E = 320000
D = 128
DH = D // 2

NC, NS = 2, 16
NW = NC * NS
EPW = E // NW
EPT = E // NS
CH = 125
ACHUNK = EPT // CH
ABUF = 4
NPAD = 10240

PPW = 2 * E // NW
PPAD = 20480
SCH = 80
SCHUNK = PPAD // SCH
SBUF = 2

_mesh = plsc.VectorSubcoreMesh(core_axis_name="c", subcore_axis_name="s")


@functools.partial(
    pl.kernel,
    out_type=(
        jax.ShapeDtypeStruct((NW, NPAD), jnp.float32),
        jax.ShapeDtypeStruct((NW, NPAD), jnp.float32),
    ),
    mesh=_mesh,
    scratch_types=[
        pltpu.VMEM((EPW,), jnp.int32),
        pltpu.VMEM((NPAD,), jnp.float32),
        pltpu.VMEM((NPAD,), jnp.float32),
    ],
    compiler_params=pltpu.CompilerParams(needs_layout_passes=False),
)
def _deg_kernel(src_hbm, dst_hbm, dout_hbm, din_hbm, idx_v, hist0, hist1):
    w = lax.axis_index("s") * NC + lax.axis_index("c")
    zeros = jnp.zeros((16,), jnp.float32)
    ones = jnp.ones((16,), jnp.float32)

    def zero_body(i, _):
        hist0[pl.ds(i * 16, 16)] = zeros
        hist1[pl.ds(i * 16, 16)] = zeros
        return 0

    lax.fori_loop(0, NPAD // 16, zero_body, 0)

    for ref, hist, out in ((src_hbm, hist0, dout_hbm), (dst_hbm, hist1, din_hbm)):
        pltpu.sync_copy(ref.at[w], idx_v)

        def acc_body(i, _, hist=hist):
            idx = idx_v[pl.ds(i * 16, 16)]
            plsc.addupdate_scatter(hist, [idx], ones)
            return 0

        lax.fori_loop(0, EPW // 16, acc_body, 0)
        pltpu.sync_copy(hist, out.at[w])


@functools.partial(
    pl.kernel,
    out_type=jax.ShapeDtypeStruct((NC, NPAD, DH), jnp.float32),
    mesh=_mesh,
    scratch_types=[
        pltpu.VMEM((ACHUNK, CH), jnp.int32),
        pltpu.VMEM((ACHUNK, CH), jnp.int32),
        pltpu.VMEM((ABUF, CH), jnp.int32),
        pltpu.VMEM((ABUF, CH, DH), jnp.float32),
        pltpu.VMEM_SHARED((NPAD, DH), jnp.float32),
        pltpu.SemaphoreType.DMA((ABUF,)),
    ],
    compiler_params=pltpu.CompilerParams(use_tc_tiling_on_sc=False),
)
def _agg_kernel(src_hbm, dst_hbm, y0_hbm, y1_hbm, out_hbm,
                sidx, didx, didx_s, rows, agg_sh, gsem):
    c = lax.axis_index("c")
    s = lax.axis_index("s")

    pltpu.sync_copy(src_hbm.at[s], sidx)
    pltpu.sync_copy(dst_hbm.at[s], didx)

    def stage_didx(b, j):
        for k in range(7):
            didx_s[b, pl.ds(k * 16, 16)] = didx[j, pl.ds(k * 16, 16)]
        didx_s[b, pl.ds(CH - 16, 16)] = didx[j, pl.ds(CH - 16, 16)]

    zeros = jnp.zeros((16,), jnp.float32)

    def zrow(i, _):
        for k in range(DH // 16):
            rows[0, i, pl.ds(k * 16, 16)] = zeros
        return 0

    lax.fori_loop(0, CH, zrow, 0)
    for k in range(8):
        pltpu.sync_copy(rows.at[0, pl.ds(0, 80)],
                        agg_sh.at[pl.ds(s * 640 + k * 80, 80)])
    plsc.subcore_barrier()

    def gather(j, b):
        @pl.when(c == 0)
        def _():
            pltpu.async_copy(y0_hbm.at[sidx.at[j]], rows.at[b], gsem.at[b])

        @pl.when(c == 1)
        def _():
            pltpu.async_copy(y1_hbm.at[sidx.at[j]], rows.at[b], gsem.at[b])

    def gwait(j, b):
        @pl.when(c == 0)
        def _():
            pltpu.make_async_copy(y0_hbm.at[sidx.at[j]], rows.at[b],
                                  gsem.at[b]).wait()

        @pl.when(c == 1)
        def _():
            pltpu.make_async_copy(y1_hbm.at[sidx.at[j]], rows.at[b],
                                  gsem.at[b]).wait()

    for b in range(ABUF):
        stage_didx(b, b)
        gather(b, b)

    def outer(jo, _):
        for b in range(ABUF):
            j = jo * ABUF + b
            gwait(j, b)
            pltpu.sync_copy(rows.at[b], agg_sh.at[didx_s.at[b]], add=True)

            @pl.when(jo < ACHUNK // ABUF - 1)
            def _():
                jn = j + ABUF
                stage_didx(b, jn)
                gather(jn, b)
        return 0

    lax.fori_loop(0, ACHUNK // ABUF, outer, 0)
    plsc.subcore_barrier()

    for k in range(8):
        sl = pl.ds(s * 640 + k * 80, 80)
        pltpu.sync_copy(agg_sh.at[sl], out_hbm.at[c, sl])


GPW = 2 * E // NW
GCH = 128
GFULL = GPW // GCH
GTAIL = GPW - GFULL * GCH
GBUF = 4


@functools.partial(
    pl.kernel,
    out_type=jax.ShapeDtypeStruct((2 * E, D), jnp.float32),
    mesh=_mesh,
    scratch_types=[
        pltpu.VMEM((GPW,), jnp.int32),
        pltpu.VMEM((GBUF, GCH, D), jnp.float32),
        pltpu.SemaphoreType.DMA((GBUF,)),
    ],
)
def _gather_kernel(gidx_hbm, h_hbm, out_hbm, sidx, rows, gsem):
    w = lax.axis_index("s") * NC + lax.axis_index("c")
    base = w * GPW
    pltpu.sync_copy(gidx_hbm.at[w], sidx)

    for b in range(GBUF):
        pltpu.async_copy(h_hbm.at[sidx.at[pl.ds(b * GCH, GCH)]], rows.at[b],
                         gsem.at[b])

    def outer(jo, _):
        for b in range(GBUF):
            j = jo * GBUF + b
            pltpu.make_async_copy(h_hbm.at[sidx.at[pl.ds(j * GCH, GCH)]],
                                  rows.at[b], gsem.at[b]).wait()
            pltpu.sync_copy(rows.at[b], out_hbm.at[pl.ds(base + j * GCH, GCH)])

            @pl.when(jo < GFULL // GBUF - 1)
            def _():
                jn = j + GBUF
                pltpu.async_copy(h_hbm.at[sidx.at[pl.ds(jn * GCH, GCH)]],
                                 rows.at[b], gsem.at[b])
        return 0

    lax.fori_loop(0, GFULL // GBUF, outer, 0)

    tsl = pl.ds(GFULL * GCH, GTAIL)
    pltpu.async_copy(h_hbm.at[sidx.at[tsl]], rows.at[0, pl.ds(0, GTAIL)],
                     gsem.at[0])
    pltpu.make_async_copy(h_hbm.at[sidx.at[tsl]], rows.at[0, pl.ds(0, GTAIL)],
                          gsem.at[0]).wait()
    pltpu.sync_copy(rows.at[0, pl.ds(0, GTAIL)],
                    out_hbm.at[pl.ds(base + GFULL * GCH, GTAIL)])


def _norm_body(dout_ref, din_ref, ns_ref, nd_ref):
    dout = jnp.sum(dout_ref[...], axis=0, keepdims=True)
    din = jnp.sum(din_ref[...], axis=0, keepdims=True)
    ns_ref[...] = jnp.where(dout > 0.0, lax.rsqrt(jnp.maximum(dout, 1.0)), 0.0)
    nd_ref[...] = jnp.where(din > 0.0, lax.rsqrt(jnp.maximum(din, 1.0)), 0.0)


def _norm_kernel(doutp, dinp):
    return pl.pallas_call(
        _norm_body,
        out_shape=(
            jax.ShapeDtypeStruct((1, NPAD), jnp.float32),
            jax.ShapeDtypeStruct((1, NPAD), jnp.float32),
        ),
    )(doutp, dinp)


BM = 2000


def _mm1_body(x_ref, ns_ref, w_ref, olo_ref, ohi_ref):
    xs = x_ref[...] * ns_ref[...]
    y = jnp.dot(xs, w_ref[...], preferred_element_type=jnp.float32)
    olo_ref[...] = y[:, :DH]
    ohi_ref[...] = y[:, DH:]


def _mm1_kernel(x, ns, w):
    return pl.pallas_call(
        _mm1_body,
        grid=(N // BM,),
        in_specs=[
            pl.BlockSpec((BM, D), lambda i: (i, 0)),
            pl.BlockSpec((BM, 1), lambda i: (i, 0)),
            pl.BlockSpec((D, D), lambda i: (0, 0)),
        ],
        out_specs=[pl.BlockSpec((BM, DH), lambda i: (i, 0))] * 2,
        out_shape=[jax.ShapeDtypeStruct((N, DH), jnp.float32)] * 2,
    )(x, ns, w)


def _mid_body(alo_ref, ahi_ref, nd_ref, b1_ref, ns_ref, w_ref,
              olo_ref, ohi_ref):
    nd = nd_ref[...]
    ns = ns_ref[...]
    hlo = alo_ref[0] * nd + b1_ref[:, :DH]
    hhi = ahi_ref[0] * nd + b1_ref[:, DH:]
    hlo = jnp.where(hlo > 0.0, hlo, 0.01 * hlo) * ns
    hhi = jnp.where(hhi > 0.0, hhi, 0.01 * hhi) * ns
    y = (
        jnp.dot(hlo, w_ref[:DH, :], preferred_element_type=jnp.float32)
        + jnp.dot(hhi, w_ref[DH:, :], preferred_element_type=jnp.float32)
    )
    olo_ref[...] = y[:, :DH]
    ohi_ref[...] = y[:, DH:]


def _mid_kernel(aggp, nd, b1r, ns, w):
    return pl.pallas_call(
        _mid_body,
        grid=(N // BM,),
        in_specs=[
            pl.BlockSpec((1, BM, DH), lambda i: (0, i, 0)),
            pl.BlockSpec((1, BM, DH), lambda i: (1, i, 0)),
            pl.BlockSpec((BM, 1), lambda i: (i, 0)),
            pl.BlockSpec((1, D), lambda i: (0, 0)),
            pl.BlockSpec((BM, 1), lambda i: (i, 0)),
            pl.BlockSpec((D, D), lambda i: (0, 0)),
        ],
        out_specs=[pl.BlockSpec((BM, DH), lambda i: (i, 0))] * 2,
        out_shape=[jax.ShapeDtypeStruct((N, DH), jnp.float32)] * 2,
    )(aggp, aggp, nd, b1r, ns, w)


def _fin_body(alo_ref, ahi_ref, nd_ref, b2_ref, o_ref):
    nd = nd_ref[...]
    hlo = alo_ref[0] * nd + b2_ref[:, :DH]
    hhi = ahi_ref[0] * nd + b2_ref[:, DH:]
    o_ref[...] = jnp.concatenate([hlo, hhi], axis=-1)


def _fin_kernel(aggp, nd, b2r):
    return pl.pallas_call(
        _fin_body,
        grid=(N // BM,),
        in_specs=[
            pl.BlockSpec((1, BM, DH), lambda i: (0, i, 0)),
            pl.BlockSpec((1, BM, DH), lambda i: (1, i, 0)),
            pl.BlockSpec((BM, 1), lambda i: (i, 0)),
            pl.BlockSpec((1, D), lambda i: (0, 0)),
        ],
        out_specs=pl.BlockSpec((BM, D), lambda i: (i, 0)),
        out_shape=jax.ShapeDtypeStruct((N, D), jnp.float32),
    )(aggp, aggp, nd, b2r)


BD = 2000


def _dot_body(a_ref, b_ref, p_ref):
    p_ref[...] = jnp.sum(a_ref[...] * b_ref[...], axis=-1, keepdims=True)


_NBLK = E // BD


def _dot_kernel(g):
    return pl.pallas_call(
        _dot_body,
        grid=(_NBLK,),
        in_specs=[
            pl.BlockSpec((BD, D), lambda i: (i, 0)),
            pl.BlockSpec((BD, D), lambda i: (i + _NBLK, 0)),
        ],
        out_specs=pl.BlockSpec((BD, 1), lambda i: (i, 0)),
        out_shape=jax.ShapeDtypeStruct((E, 1), jnp.float32),
    )(g, g)


@jax.jit
def kernel(edge_index, neg_edge_index, embed, W1, b1, W2, b2):
    src = edge_index[0]
    dst = edge_index[1]

    srcw = src.reshape(NW, EPW)
    dstw = dst.reshape(NW, EPW)
    srct = src.reshape(NS, ACHUNK, CH)
    dstt = dst.reshape(NS, ACHUNK, CH)

    doutp, dinp = _deg_kernel(srcw, dstw)
    nso, ndo = _norm_kernel(doutp, dinp)
    ns = nso[0, :N].reshape(N, 1)
    nd = ndo[0, :N].reshape(N, 1)

    y1lo, y1hi = _mm1_kernel(embed, ns, W1)
    aggp1 = _agg_kernel(srct, dstt, y1lo, y1hi)
    y2lo, y2hi = _mid_kernel(aggp1, nd, b1.reshape(1, D), ns, W2)
    aggp2 = _agg_kernel(srct, dstt, y2lo, y2hi)
    h = _fin_kernel(aggp2, nd, b2.reshape(1, D))

    gidxp = jnp.concatenate([src, dst]).reshape(NW, GPW)
    gidxn = jnp.concatenate(
        [neg_edge_index[0], neg_edge_index[1]]).reshape(NW, GPW)
    gp = _gather_kernel(gidxp, h)
    gn = _gather_kernel(gidxn, h)
    pos = _dot_kernel(gp)
    neg = _dot_kernel(gn)
    return pos, neg


# speedup vs baseline: 1.7897x; 1.0360x over previous
import functools
import jax
import jax.numpy as jnp
from jax import lax
from jax.experimental import pallas as pl
from jax.experimental.pallas import tpu as pltpu
from jax.experimental.pallas import tpu_sc as plsc

N = 10000
E = 320000
D = 128
DH = D // 2

NC, NS = 2, 16
NW = NC * NS
EPW = E // NW
EPT = E // NS
CH = 125
ACHUNK = EPT // CH
ABUF = 4
NPAD = 10240

PPW = 2 * E // NW
PPAD = 20480
SCH = 80
SCHUNK = PPAD // SCH
SBUF = 2

_mesh = plsc.VectorSubcoreMesh(core_axis_name="c", subcore_axis_name="s")


@functools.partial(
    pl.kernel,
    out_type=(
        jax.ShapeDtypeStruct((NW, NPAD), jnp.float32),
        jax.ShapeDtypeStruct((NW, NPAD), jnp.float32),
    ),
    mesh=_mesh,
    scratch_types=[
        pltpu.VMEM((EPW,), jnp.int32),
        pltpu.VMEM((NPAD,), jnp.float32),
        pltpu.VMEM((NPAD,), jnp.float32),
    ],
    compiler_params=pltpu.CompilerParams(needs_layout_passes=False),
)
def _deg_kernel(src_hbm, dst_hbm, dout_hbm, din_hbm, idx_v, hist0, hist1):
    w = lax.axis_index("s") * NC + lax.axis_index("c")
    zeros = jnp.zeros((16,), jnp.float32)
    ones = jnp.ones((16,), jnp.float32)

    def zero_body(i, _):
        hist0[pl.ds(i * 16, 16)] = zeros
        hist1[pl.ds(i * 16, 16)] = zeros
        return 0

    lax.fori_loop(0, NPAD // 16, zero_body, 0)

    for ref, hist, out in ((src_hbm, hist0, dout_hbm), (dst_hbm, hist1, din_hbm)):
        pltpu.sync_copy(ref.at[w], idx_v)

        def acc_body(i, _, hist=hist):
            idx = idx_v[pl.ds(i * 16, 16)]
            plsc.addupdate_scatter(hist, [idx], ones)
            return 0

        lax.fori_loop(0, EPW // 16, acc_body, 0)
        pltpu.sync_copy(hist, out.at[w])


@functools.partial(
    pl.kernel,
    out_type=jax.ShapeDtypeStruct((NC, NPAD, DH), jnp.float32),
    mesh=_mesh,
    scratch_types=[
        pltpu.VMEM((ACHUNK, CH), jnp.int32),
        pltpu.VMEM((ACHUNK, CH), jnp.int32),
        pltpu.VMEM((ABUF, CH), jnp.int32),
        pltpu.VMEM((ABUF, CH, DH), jnp.float32),
        pltpu.VMEM_SHARED((NPAD, DH), jnp.float32),
        pltpu.SemaphoreType.DMA((ABUF,)),
    ],
    compiler_params=pltpu.CompilerParams(use_tc_tiling_on_sc=False),
)
def _agg_kernel(src_hbm, dst_hbm, y0_hbm, y1_hbm, out_hbm,
                sidx, didx, didx_s, rows, agg_sh, gsem):
    c = lax.axis_index("c")
    s = lax.axis_index("s")

    pltpu.sync_copy(src_hbm.at[s], sidx)
    pltpu.sync_copy(dst_hbm.at[s], didx)

    def stage_didx(b, j):
        for k in range(7):
            didx_s[b, pl.ds(k * 16, 16)] = didx[j, pl.ds(k * 16, 16)]
        didx_s[b, pl.ds(CH - 16, 16)] = didx[j, pl.ds(CH - 16, 16)]

    zeros = jnp.zeros((16,), jnp.float32)

    def zrow(i, _):
        for k in range(DH // 16):
            rows[0, i, pl.ds(k * 16, 16)] = zeros
        return 0

    lax.fori_loop(0, CH, zrow, 0)
    for k in range(8):
        pltpu.sync_copy(rows.at[0, pl.ds(0, 80)],
                        agg_sh.at[pl.ds(s * 640 + k * 80, 80)])
    plsc.subcore_barrier()

    def gather(j, b):
        @pl.when(c == 0)
        def _():
            pltpu.async_copy(y0_hbm.at[sidx.at[j]], rows.at[b], gsem.at[b])

        @pl.when(c == 1)
        def _():
            pltpu.async_copy(y1_hbm.at[sidx.at[j]], rows.at[b], gsem.at[b])

    def gwait(j, b):
        @pl.when(c == 0)
        def _():
            pltpu.make_async_copy(y0_hbm.at[sidx.at[j]], rows.at[b],
                                  gsem.at[b]).wait()

        @pl.when(c == 1)
        def _():
            pltpu.make_async_copy(y1_hbm.at[sidx.at[j]], rows.at[b],
                                  gsem.at[b]).wait()

    for b in range(ABUF):
        stage_didx(b, b)
        gather(b, b)

    def outer(jo, _):
        for b in range(ABUF):
            j = jo * ABUF + b
            gwait(j, b)
            pltpu.sync_copy(rows.at[b], agg_sh.at[didx_s.at[b]], add=True)

            @pl.when(jo < ACHUNK // ABUF - 1)
            def _():
                jn = j + ABUF
                stage_didx(b, jn)
                gather(jn, b)
        return 0

    lax.fori_loop(0, ACHUNK // ABUF, outer, 0)
    plsc.subcore_barrier()

    for k in range(8):
        sl = pl.ds(s * 640 + k * 80, 80)
        pltpu.sync_copy(agg_sh.at[sl], out_hbm.at[c, sl])


GPW = 4 * E // NW
GCH = 128
GFULL = GPW // GCH
GTAIL = GPW - GFULL * GCH
GBUF = 4


@functools.partial(
    pl.kernel,
    out_type=jax.ShapeDtypeStruct((4 * E, D), jnp.float32),
    mesh=_mesh,
    scratch_types=[
        pltpu.VMEM((GPW,), jnp.int32),
        pltpu.VMEM((GBUF, GCH, D), jnp.float32),
        pltpu.SemaphoreType.DMA((GBUF,)),
    ],
)
def _gather_kernel(gidx_hbm, h_hbm, out_hbm, sidx, rows, gsem):
    w = lax.axis_index("s") * NC + lax.axis_index("c")
    base = w * GPW
    pltpu.sync_copy(gidx_hbm.at[w], sidx)

    for b in range(GBUF):
        pltpu.async_copy(h_hbm.at[sidx.at[pl.ds(b * GCH, GCH)]], rows.at[b],
                         gsem.at[b])

    def outer(jo, _):
        for b in range(GBUF):
            j = jo * GBUF + b
            pltpu.make_async_copy(h_hbm.at[sidx.at[pl.ds(j * GCH, GCH)]],
                                  rows.at[b], gsem.at[b]).wait()
            pltpu.sync_copy(rows.at[b], out_hbm.at[pl.ds(base + j * GCH, GCH)])

            @pl.when(jo < GFULL // GBUF - 1)
            def _():
                jn = j + GBUF
                pltpu.async_copy(h_hbm.at[sidx.at[pl.ds(jn * GCH, GCH)]],
                                 rows.at[b], gsem.at[b])
        return 0

    lax.fori_loop(0, GFULL // GBUF, outer, 0)

    tsl = pl.ds(GFULL * GCH, GTAIL)
    pltpu.async_copy(h_hbm.at[sidx.at[tsl]], rows.at[0, pl.ds(0, GTAIL)],
                     gsem.at[0])
    pltpu.make_async_copy(h_hbm.at[sidx.at[tsl]], rows.at[0, pl.ds(0, GTAIL)],
                          gsem.at[0]).wait()
    pltpu.sync_copy(rows.at[0, pl.ds(0, GTAIL)],
                    out_hbm.at[pl.ds(base + GFULL * GCH, GTAIL)])


def _norm_body(dout_ref, din_ref, ns_ref, nd_ref):
    dout = jnp.sum(dout_ref[...], axis=0, keepdims=True)
    din = jnp.sum(din_ref[...], axis=0, keepdims=True)
    ns_ref[...] = jnp.where(dout > 0.0, lax.rsqrt(jnp.maximum(dout, 1.0)), 0.0)
    nd_ref[...] = jnp.where(din > 0.0, lax.rsqrt(jnp.maximum(din, 1.0)), 0.0)


def _norm_kernel(doutp, dinp):
    return pl.pallas_call(
        _norm_body,
        out_shape=(
            jax.ShapeDtypeStruct((1, NPAD), jnp.float32),
            jax.ShapeDtypeStruct((1, NPAD), jnp.float32),
        ),
    )(doutp, dinp)


BM = 2000


def _mm1_body(x_ref, ns_ref, w_ref, olo_ref, ohi_ref):
    xs = x_ref[...] * ns_ref[...]
    y = jnp.dot(xs, w_ref[...], preferred_element_type=jnp.float32)
    olo_ref[...] = y[:, :DH]
    ohi_ref[...] = y[:, DH:]


def _mm1_kernel(x, ns, w):
    return pl.pallas_call(
        _mm1_body,
        grid=(N // BM,),
        in_specs=[
            pl.BlockSpec((BM, D), lambda i: (i, 0)),
            pl.BlockSpec((BM, 1), lambda i: (i, 0)),
            pl.BlockSpec((D, D), lambda i: (0, 0)),
        ],
        out_specs=[pl.BlockSpec((BM, DH), lambda i: (i, 0))] * 2,
        out_shape=[jax.ShapeDtypeStruct((N, DH), jnp.float32)] * 2,
    )(x, ns, w)


def _mid_body(alo_ref, ahi_ref, nd_ref, b1_ref, ns_ref, w_ref,
              olo_ref, ohi_ref):
    nd = nd_ref[...]
    ns = ns_ref[...]
    hlo = alo_ref[0] * nd + b1_ref[:, :DH]
    hhi = ahi_ref[0] * nd + b1_ref[:, DH:]
    hlo = jnp.where(hlo > 0.0, hlo, 0.01 * hlo) * ns
    hhi = jnp.where(hhi > 0.0, hhi, 0.01 * hhi) * ns
    y = (
        jnp.dot(hlo, w_ref[:DH, :], preferred_element_type=jnp.float32)
        + jnp.dot(hhi, w_ref[DH:, :], preferred_element_type=jnp.float32)
    )
    olo_ref[...] = y[:, :DH]
    ohi_ref[...] = y[:, DH:]


def _mid_kernel(aggp, nd, b1r, ns, w):
    return pl.pallas_call(
        _mid_body,
        grid=(N // BM,),
        in_specs=[
            pl.BlockSpec((1, BM, DH), lambda i: (0, i, 0)),
            pl.BlockSpec((1, BM, DH), lambda i: (1, i, 0)),
            pl.BlockSpec((BM, 1), lambda i: (i, 0)),
            pl.BlockSpec((1, D), lambda i: (0, 0)),
            pl.BlockSpec((BM, 1), lambda i: (i, 0)),
            pl.BlockSpec((D, D), lambda i: (0, 0)),
        ],
        out_specs=[pl.BlockSpec((BM, DH), lambda i: (i, 0))] * 2,
        out_shape=[jax.ShapeDtypeStruct((N, DH), jnp.float32)] * 2,
    )(aggp, aggp, nd, b1r, ns, w)


def _fin_body(alo_ref, ahi_ref, nd_ref, b2_ref, o_ref):
    nd = nd_ref[...]
    hlo = alo_ref[0] * nd + b2_ref[:, :DH]
    hhi = ahi_ref[0] * nd + b2_ref[:, DH:]
    o_ref[...] = jnp.concatenate([hlo, hhi], axis=-1)


def _fin_kernel(aggp, nd, b2r):
    return pl.pallas_call(
        _fin_body,
        grid=(N // BM,),
        in_specs=[
            pl.BlockSpec((1, BM, DH), lambda i: (0, i, 0)),
            pl.BlockSpec((1, BM, DH), lambda i: (1, i, 0)),
            pl.BlockSpec((BM, 1), lambda i: (i, 0)),
            pl.BlockSpec((1, D), lambda i: (0, 0)),
        ],
        out_specs=pl.BlockSpec((BM, D), lambda i: (i, 0)),
        out_shape=jax.ShapeDtypeStruct((N, D), jnp.float32),
    )(aggp, aggp, nd, b2r)


BD = 4000


def _dot_body(a_ref, b_ref, c_ref, d_ref, p_ref, n_ref):
    a = a_ref[...].astype(jnp.float32)
    b = b_ref[...].astype(jnp.float32)
    c = c_ref[...].astype(jnp.float32)
    d = d_ref[...].astype(jnp.float32)
    p_ref[...] = jnp.sum(a * b, axis=-1, keepdims=True)
    n_ref[...] = jnp.sum(c * d, axis=-1, keepdims=True)


_NBLK = E // BD


def _dot_kernel(g):
    return pl.pallas_call(
        _dot_body,
        grid=(_NBLK,),
        in_specs=[
            pl.BlockSpec((BD, D), lambda i: (i, 0)),
            pl.BlockSpec((BD, D), lambda i: (i + _NBLK, 0)),
            pl.BlockSpec((BD, D), lambda i: (i + 2 * _NBLK, 0)),
            pl.BlockSpec((BD, D), lambda i: (i + 3 * _NBLK, 0)),
        ],
        out_specs=[pl.BlockSpec((BD, 1), lambda i: (i, 0))] * 2,
        out_shape=[jax.ShapeDtypeStruct((E, 1), jnp.float32)] * 2,
    )(g, g, g, g)


@jax.jit
def kernel(edge_index, neg_edge_index, embed, W1, b1, W2, b2):
    src = edge_index[0]
    dst = edge_index[1]

    srcw = src.reshape(NW, EPW)
    dstw = dst.reshape(NW, EPW)
    srct = src.reshape(NS, ACHUNK, CH)
    dstt = dst.reshape(NS, ACHUNK, CH)

    doutp, dinp = _deg_kernel(srcw, dstw)
    nso, ndo = _norm_kernel(doutp, dinp)
    ns = nso[0, :N].reshape(N, 1)
    nd = ndo[0, :N].reshape(N, 1)

    y1lo, y1hi = _mm1_kernel(embed, ns, W1)
    aggp1 = _agg_kernel(srct, dstt, y1lo, y1hi)
    y2lo, y2hi = _mid_kernel(aggp1, nd, b1.reshape(1, D), ns, W2)
    aggp2 = _agg_kernel(srct, dstt, y2lo, y2hi)
    h = _fin_kernel(aggp2, nd, b2.reshape(1, D))

    gidx = jnp.concatenate(
        [src, dst, neg_edge_index[0], neg_edge_index[1]]
    ).reshape(NW, GPW)
    g = _gather_kernel(gidx, h)

    pos, neg = _dot_kernel(g)
    return pos, neg


# speedup vs baseline: 1.7904x; 1.0004x over previous
import functools
import jax
import jax.numpy as jnp
from jax import lax
from jax.experimental import pallas as pl
from jax.experimental.pallas import tpu as pltpu
from jax.experimental.pallas import tpu_sc as plsc

N = 10000
E = 320000
D = 128
DH = D // 2

NC, NS = 2, 16
NW = NC * NS
EPW = E // NW
EPT = E // NS
CH = 125
ACHUNK = EPT // CH
ABUF = 4
NPAD = 10240

_mesh = plsc.VectorSubcoreMesh(core_axis_name="c", subcore_axis_name="s")


@functools.partial(
    pl.kernel,
    out_type=(
        jax.ShapeDtypeStruct((NW, NPAD), jnp.float32),
        jax.ShapeDtypeStruct((NW, NPAD), jnp.float32),
    ),
    mesh=_mesh,
    scratch_types=[
        pltpu.VMEM((EPW,), jnp.int32),
        pltpu.VMEM((NPAD,), jnp.float32),
        pltpu.VMEM((NPAD,), jnp.float32),
    ],
    compiler_params=pltpu.CompilerParams(needs_layout_passes=False),
)
def _deg_kernel(src_hbm, dst_hbm, dout_hbm, din_hbm, idx_v, hist0, hist1):
    w = lax.axis_index("s") * NC + lax.axis_index("c")
    zeros = jnp.zeros((16,), jnp.float32)
    ones = jnp.ones((16,), jnp.float32)

    def zero_body(i, _):
        hist0[pl.ds(i * 16, 16)] = zeros
        hist1[pl.ds(i * 16, 16)] = zeros
        return 0

    lax.fori_loop(0, NPAD // 16, zero_body, 0)

    for ref, hist, out in ((src_hbm, hist0, dout_hbm), (dst_hbm, hist1, din_hbm)):
        pltpu.sync_copy(ref.at[w], idx_v)

        def acc_body(i, _, hist=hist):
            idx = idx_v[pl.ds(i * 16, 16)]
            plsc.addupdate_scatter(hist, [idx], ones)
            return 0

        lax.fori_loop(0, EPW // 16, acc_body, 0)
        pltpu.sync_copy(hist, out.at[w])


@functools.partial(
    pl.kernel,
    out_type=jax.ShapeDtypeStruct((NC, NPAD, DH), jnp.float32),
    mesh=_mesh,
    scratch_types=[
        pltpu.VMEM((ACHUNK, CH), jnp.int32),
        pltpu.VMEM((ACHUNK, CH), jnp.int32),
        pltpu.VMEM((ABUF, CH), jnp.int32),
        pltpu.VMEM((ABUF, CH, DH), jnp.float32),
        pltpu.VMEM_SHARED((NPAD, DH), jnp.float32),
        pltpu.SemaphoreType.DMA((ABUF,)),
    ],
    compiler_params=pltpu.CompilerParams(use_tc_tiling_on_sc=False),
)
def _agg_kernel(src_hbm, dst_hbm, y0_hbm, y1_hbm, out_hbm,
                sidx, didx, didx_s, rows, agg_sh, gsem):
    c = lax.axis_index("c")
    s = lax.axis_index("s")

    pltpu.sync_copy(src_hbm.at[s], sidx)
    pltpu.sync_copy(dst_hbm.at[s], didx)

    def stage_didx(b, j):
        for k in range(7):
            didx_s[b, pl.ds(k * 16, 16)] = didx[j, pl.ds(k * 16, 16)]
        didx_s[b, pl.ds(CH - 16, 16)] = didx[j, pl.ds(CH - 16, 16)]

    zeros = jnp.zeros((16,), jnp.float32)

    def zrow(i, _):
        for k in range(DH // 16):
            rows[0, i, pl.ds(k * 16, 16)] = zeros
        return 0

    lax.fori_loop(0, CH, zrow, 0)
    for k in range(8):
        pltpu.sync_copy(rows.at[0, pl.ds(0, 80)],
                        agg_sh.at[pl.ds(s * 640 + k * 80, 80)])
    plsc.subcore_barrier()

    def gather(j, b):
        @pl.when(c == 0)
        def _():
            pltpu.async_copy(y0_hbm.at[sidx.at[j]], rows.at[b], gsem.at[b])

        @pl.when(c == 1)
        def _():
            pltpu.async_copy(y1_hbm.at[sidx.at[j]], rows.at[b], gsem.at[b])

    def gwait(j, b):
        @pl.when(c == 0)
        def _():
            pltpu.make_async_copy(y0_hbm.at[sidx.at[j]], rows.at[b],
                                  gsem.at[b]).wait()

        @pl.when(c == 1)
        def _():
            pltpu.make_async_copy(y1_hbm.at[sidx.at[j]], rows.at[b],
                                  gsem.at[b]).wait()

    for b in range(ABUF):
        stage_didx(b, b)
        gather(b, b)

    def outer(jo, _):
        for b in range(ABUF):
            j = jo * ABUF + b
            gwait(j, b)
            pltpu.sync_copy(rows.at[b], agg_sh.at[didx_s.at[b]], add=True)

            @pl.when(jo < ACHUNK // ABUF - 1)
            def _():
                jn = j + ABUF
                stage_didx(b, jn)
                gather(jn, b)
        return 0

    lax.fori_loop(0, ACHUNK // ABUF, outer, 0)
    plsc.subcore_barrier()

    for k in range(8):
        sl = pl.ds(s * 640 + k * 80, 80)
        pltpu.sync_copy(agg_sh.at[sl], out_hbm.at[c, sl])


GPW = 4 * E // NW
GCH = 128
GFULL = GPW // GCH
GTAIL = GPW - GFULL * GCH
GBUF = 4


@functools.partial(
    pl.kernel,
    out_type=jax.ShapeDtypeStruct((4 * E, D), jnp.float32),
    mesh=_mesh,
    scratch_types=[
        pltpu.VMEM((GPW,), jnp.int32),
        pltpu.VMEM((GBUF, GCH, D), jnp.float32),
        pltpu.SemaphoreType.DMA((GBUF,)),
    ],
)
def _gather_kernel(gidx_hbm, h_hbm, out_hbm, sidx, rows, gsem):
    w = lax.axis_index("s") * NC + lax.axis_index("c")
    base = w * GPW
    pltpu.sync_copy(gidx_hbm.at[w], sidx)

    for b in range(GBUF):
        pltpu.async_copy(h_hbm.at[sidx.at[pl.ds(b * GCH, GCH)]], rows.at[b],
                         gsem.at[b])

    def outer(jo, _):
        for b in range(GBUF):
            j = jo * GBUF + b
            pltpu.make_async_copy(h_hbm.at[sidx.at[pl.ds(j * GCH, GCH)]],
                                  rows.at[b], gsem.at[b]).wait()
            pltpu.sync_copy(rows.at[b], out_hbm.at[pl.ds(base + j * GCH, GCH)])

            @pl.when(jo < GFULL // GBUF - 1)
            def _():
                jn = j + GBUF
                pltpu.async_copy(h_hbm.at[sidx.at[pl.ds(jn * GCH, GCH)]],
                                 rows.at[b], gsem.at[b])
        return 0

    lax.fori_loop(0, GFULL // GBUF, outer, 0)

    tsl = pl.ds(GFULL * GCH, GTAIL)
    pltpu.async_copy(h_hbm.at[sidx.at[tsl]], rows.at[0, pl.ds(0, GTAIL)],
                     gsem.at[0])
    pltpu.make_async_copy(h_hbm.at[sidx.at[tsl]], rows.at[0, pl.ds(0, GTAIL)],
                          gsem.at[0]).wait()
    pltpu.sync_copy(rows.at[0, pl.ds(0, GTAIL)],
                    out_hbm.at[pl.ds(base + GFULL * GCH, GTAIL)])


def _norm_body(dout_ref, din_ref, ns_ref, nd_ref):
    dout = jnp.sum(dout_ref[...], axis=0, keepdims=True)
    din = jnp.sum(din_ref[...], axis=0, keepdims=True)
    ns_ref[...] = jnp.where(dout > 0.0, lax.rsqrt(jnp.maximum(dout, 1.0)), 0.0)
    nd_ref[...] = jnp.where(din > 0.0, lax.rsqrt(jnp.maximum(din, 1.0)), 0.0)


def _norm_kernel(doutp, dinp):
    return pl.pallas_call(
        _norm_body,
        out_shape=(
            jax.ShapeDtypeStruct((1, NPAD), jnp.float32),
            jax.ShapeDtypeStruct((1, NPAD), jnp.float32),
        ),
    )(doutp, dinp)


BM = 2000


def _mm1_body(x_ref, ns_ref, w_ref, olo_ref, ohi_ref):
    xs = x_ref[...] * ns_ref[...]
    y = jnp.dot(xs, w_ref[...], preferred_element_type=jnp.float32)
    olo_ref[...] = y[:, :DH]
    ohi_ref[...] = y[:, DH:]


def _mm1_kernel(x, ns, w):
    return pl.pallas_call(
        _mm1_body,
        grid=(N // BM,),
        in_specs=[
            pl.BlockSpec((BM, D), lambda i: (i, 0)),
            pl.BlockSpec((BM, 1), lambda i: (i, 0)),
            pl.BlockSpec((D, D), lambda i: (0, 0)),
        ],
        out_specs=[pl.BlockSpec((BM, DH), lambda i: (i, 0))] * 2,
        out_shape=[jax.ShapeDtypeStruct((N, DH), jnp.float32)] * 2,
    )(x, ns, w)


def _mid_body(alo_ref, ahi_ref, nd_ref, b1_ref, ns_ref, w_ref,
              olo_ref, ohi_ref):
    nd = nd_ref[...]
    ns = ns_ref[...]
    hlo = alo_ref[0] * nd + b1_ref[:, :DH]
    hhi = ahi_ref[0] * nd + b1_ref[:, DH:]
    hlo = jnp.where(hlo > 0.0, hlo, 0.01 * hlo) * ns
    hhi = jnp.where(hhi > 0.0, hhi, 0.01 * hhi) * ns
    y = (
        jnp.dot(hlo, w_ref[:DH, :], preferred_element_type=jnp.float32)
        + jnp.dot(hhi, w_ref[DH:, :], preferred_element_type=jnp.float32)
    )
    olo_ref[...] = y[:, :DH]
    ohi_ref[...] = y[:, DH:]


def _mid_kernel(aggp, nd, b1r, ns, w):
    return pl.pallas_call(
        _mid_body,
        grid=(N // BM,),
        in_specs=[
            pl.BlockSpec((1, BM, DH), lambda i: (0, i, 0)),
            pl.BlockSpec((1, BM, DH), lambda i: (1, i, 0)),
            pl.BlockSpec((BM, 1), lambda i: (i, 0)),
            pl.BlockSpec((1, D), lambda i: (0, 0)),
            pl.BlockSpec((BM, 1), lambda i: (i, 0)),
            pl.BlockSpec((D, D), lambda i: (0, 0)),
        ],
        out_specs=[pl.BlockSpec((BM, DH), lambda i: (i, 0))] * 2,
        out_shape=[jax.ShapeDtypeStruct((N, DH), jnp.float32)] * 2,
    )(aggp, aggp, nd, b1r, ns, w)


def _fin_body(alo_ref, ahi_ref, nd_ref, b2_ref, o_ref):
    nd = nd_ref[...]
    hlo = alo_ref[0] * nd + b2_ref[:, :DH]
    hhi = ahi_ref[0] * nd + b2_ref[:, DH:]
    o_ref[...] = jnp.concatenate([hlo, hhi], axis=-1)


def _fin_kernel(aggp, nd, b2r):
    return pl.pallas_call(
        _fin_body,
        grid=(N // BM,),
        in_specs=[
            pl.BlockSpec((1, BM, DH), lambda i: (0, i, 0)),
            pl.BlockSpec((1, BM, DH), lambda i: (1, i, 0)),
            pl.BlockSpec((BM, 1), lambda i: (i, 0)),
            pl.BlockSpec((1, D), lambda i: (0, 0)),
        ],
        out_specs=pl.BlockSpec((BM, D), lambda i: (i, 0)),
        out_shape=jax.ShapeDtypeStruct((N, D), jnp.float32),
    )(aggp, aggp, nd, b2r)


BD = 4000


def _dot_body(a_ref, b_ref, c_ref, d_ref, p_ref, n_ref):
    a = a_ref[...].astype(jnp.float32)
    b = b_ref[...].astype(jnp.float32)
    c = c_ref[...].astype(jnp.float32)
    d = d_ref[...].astype(jnp.float32)
    p_ref[...] = jnp.sum(a * b, axis=-1, keepdims=True)
    n_ref[...] = jnp.sum(c * d, axis=-1, keepdims=True)


_NBLK = E // BD


def _dot_kernel(g):
    return pl.pallas_call(
        _dot_body,
        grid=(_NBLK,),
        in_specs=[
            pl.BlockSpec((BD, D), lambda i: (i, 0)),
            pl.BlockSpec((BD, D), lambda i: (i + _NBLK, 0)),
            pl.BlockSpec((BD, D), lambda i: (i + 2 * _NBLK, 0)),
            pl.BlockSpec((BD, D), lambda i: (i + 3 * _NBLK, 0)),
        ],
        out_specs=[pl.BlockSpec((BD, 1), lambda i: (i, 0))] * 2,
        out_shape=[jax.ShapeDtypeStruct((E, 1), jnp.float32)] * 2,
    )(g, g, g, g)


@jax.jit
def kernel(edge_index, neg_edge_index, embed, W1, b1, W2, b2):
    src = edge_index[0]
    dst = edge_index[1]

    srcw = src.reshape(NW, EPW)
    dstw = dst.reshape(NW, EPW)
    srct = src.reshape(NS, ACHUNK, CH)
    dstt = dst.reshape(NS, ACHUNK, CH)

    doutp, dinp = _deg_kernel(srcw, dstw)
    nso, ndo = _norm_kernel(doutp, dinp)
    ns = nso[0, :N].reshape(N, 1)
    nd = ndo[0, :N].reshape(N, 1)

    y1lo, y1hi = _mm1_kernel(embed, ns, W1)
    aggp1 = _agg_kernel(srct, dstt, y1lo, y1hi)
    y2lo, y2hi = _mid_kernel(aggp1, nd, b1.reshape(1, D), ns, W2)
    aggp2 = _agg_kernel(srct, dstt, y2lo, y2hi)
    h = _fin_kernel(aggp2, nd, b2.reshape(1, D))

    gidx = jnp.concatenate(
        [src, dst, neg_edge_index[0], neg_edge_index[1]]
    ).reshape(NW, GPW)
    g = _gather_kernel(gidx, h)

    pos, neg = _dot_kernel(g)
    return pos, neg


# speedup vs baseline: 1.7921x; 1.0010x over previous
import functools
import jax
import jax.numpy as jnp
from jax import lax
from jax.experimental import pallas as pl
from jax.experimental.pallas import tpu as pltpu
from jax.experimental.pallas import tpu_sc as plsc

N = 10000
E = 320000
D = 128
DH = D // 2

NC, NS = 2, 16
NW = NC * NS
EPW = E // NW
EPT = E // NS
CH = 125
ACHUNK = EPT // CH
ABUF = 4
NPAD = 10240

_mesh = plsc.VectorSubcoreMesh(core_axis_name="c", subcore_axis_name="s")


@functools.partial(
    pl.kernel,
    out_type=(
        jax.ShapeDtypeStruct((NW, NPAD), jnp.float32),
        jax.ShapeDtypeStruct((NW, NPAD), jnp.float32),
    ),
    mesh=_mesh,
    scratch_types=[
        pltpu.VMEM((EPW,), jnp.int32),
        pltpu.VMEM((NPAD,), jnp.float32),
        pltpu.VMEM((NPAD,), jnp.float32),
    ],
    compiler_params=pltpu.CompilerParams(needs_layout_passes=False),
)
def _deg_kernel(src_hbm, dst_hbm, dout_hbm, din_hbm, idx_v, hist0, hist1):
    w = lax.axis_index("s") * NC + lax.axis_index("c")
    zeros = jnp.zeros((16,), jnp.float32)
    ones = jnp.ones((16,), jnp.float32)

    def zero_body(i, _):
        hist0[pl.ds(i * 16, 16)] = zeros
        hist1[pl.ds(i * 16, 16)] = zeros
        return 0

    lax.fori_loop(0, NPAD // 16, zero_body, 0)

    for ref, hist, out in ((src_hbm, hist0, dout_hbm), (dst_hbm, hist1, din_hbm)):
        pltpu.sync_copy(ref.at[w], idx_v)

        def acc_body(i, _, hist=hist):
            idx = idx_v[pl.ds(i * 16, 16)]
            plsc.addupdate_scatter(hist, [idx], ones)
            return 0

        lax.fori_loop(0, EPW // 16, acc_body, 0)
        pltpu.sync_copy(hist, out.at[w])


@functools.partial(
    pl.kernel,
    out_type=jax.ShapeDtypeStruct((NC, NPAD, DH), jnp.float32),
    mesh=_mesh,
    scratch_types=[
        pltpu.VMEM((ACHUNK, CH), jnp.int32),
        pltpu.VMEM((ACHUNK, CH), jnp.int32),
        pltpu.VMEM((ABUF, CH), jnp.int32),
        pltpu.VMEM((ABUF, CH, DH), jnp.float32),
        pltpu.VMEM_SHARED((NPAD, DH), jnp.float32),
        pltpu.SemaphoreType.DMA((ABUF,)),
    ],
    compiler_params=pltpu.CompilerParams(use_tc_tiling_on_sc=False),
)
def _agg_kernel(src_hbm, dst_hbm, y0_hbm, y1_hbm, out_hbm,
                sidx, didx, didx_s, rows, agg_sh, gsem):
    c = lax.axis_index("c")
    s = lax.axis_index("s")

    pltpu.sync_copy(src_hbm.at[s], sidx)
    pltpu.sync_copy(dst_hbm.at[s], didx)

    def stage_didx(b, j):
        for k in range(7):
            didx_s[b, pl.ds(k * 16, 16)] = didx[j, pl.ds(k * 16, 16)]
        didx_s[b, pl.ds(CH - 16, 16)] = didx[j, pl.ds(CH - 16, 16)]

    zeros = jnp.zeros((16,), jnp.float32)

    def zrow(i, _):
        for k in range(DH // 16):
            rows[0, i, pl.ds(k * 16, 16)] = zeros
        return 0

    lax.fori_loop(0, CH, zrow, 0)
    for k in range(8):
        pltpu.sync_copy(rows.at[0, pl.ds(0, 80)],
                        agg_sh.at[pl.ds(s * 640 + k * 80, 80)])
    plsc.subcore_barrier()

    def gather(j, b):
        @pl.when(c == 0)
        def _():
            pltpu.async_copy(y0_hbm.at[sidx.at[j]], rows.at[b], gsem.at[b])

        @pl.when(c == 1)
        def _():
            pltpu.async_copy(y1_hbm.at[sidx.at[j]], rows.at[b], gsem.at[b])

    def gwait(j, b):
        @pl.when(c == 0)
        def _():
            pltpu.make_async_copy(y0_hbm.at[sidx.at[j]], rows.at[b],
                                  gsem.at[b]).wait()

        @pl.when(c == 1)
        def _():
            pltpu.make_async_copy(y1_hbm.at[sidx.at[j]], rows.at[b],
                                  gsem.at[b]).wait()

    for b in range(ABUF):
        stage_didx(b, b)
        gather(b, b)

    def outer(jo, _):
        for b in range(ABUF):
            j = jo * ABUF + b
            gwait(j, b)
            pltpu.sync_copy(rows.at[b], agg_sh.at[didx_s.at[b]], add=True)

            @pl.when(jo < ACHUNK // ABUF - 1)
            def _():
                jn = j + ABUF
                stage_didx(b, jn)
                gather(jn, b)
        return 0

    lax.fori_loop(0, ACHUNK // ABUF, outer, 0)
    plsc.subcore_barrier()

    for k in range(8):
        sl = pl.ds(s * 640 + k * 80, 80)
        pltpu.sync_copy(agg_sh.at[sl], out_hbm.at[c, sl])


GPW = 4 * E // NW
GCH = 128
GFULL = GPW // GCH
GTAIL = GPW - GFULL * GCH
GBUF = 4


@functools.partial(
    pl.kernel,
    out_type=jax.ShapeDtypeStruct((4 * E, D), jnp.float32),
    mesh=_mesh,
    scratch_types=[
        pltpu.VMEM((GPW,), jnp.int32),
        pltpu.VMEM((GBUF, GCH, D), jnp.float32),
        pltpu.SemaphoreType.DMA((GBUF,)),
    ],
)
def _gather_kernel(gidx_hbm, h_hbm, out_hbm, sidx, rows, gsem):
    w = lax.axis_index("s") * NC + lax.axis_index("c")
    base = w * GPW
    pltpu.sync_copy(gidx_hbm.at[w], sidx)

    for b in range(GBUF):
        pltpu.async_copy(h_hbm.at[sidx.at[pl.ds(b * GCH, GCH)]], rows.at[b],
                         gsem.at[b])

    def outer(jo, _):
        for b in range(GBUF):
            j = jo * GBUF + b
            pltpu.make_async_copy(h_hbm.at[sidx.at[pl.ds(j * GCH, GCH)]],
                                  rows.at[b], gsem.at[b]).wait()
            pltpu.sync_copy(rows.at[b], out_hbm.at[pl.ds(base + j * GCH, GCH)])

            @pl.when(jo < GFULL // GBUF - 1)
            def _():
                jn = j + GBUF
                pltpu.async_copy(h_hbm.at[sidx.at[pl.ds(jn * GCH, GCH)]],
                                 rows.at[b], gsem.at[b])
        return 0

    lax.fori_loop(0, GFULL // GBUF, outer, 0)

    tsl = pl.ds(GFULL * GCH, GTAIL)
    pltpu.async_copy(h_hbm.at[sidx.at[tsl]], rows.at[0, pl.ds(0, GTAIL)],
                     gsem.at[0])
    pltpu.make_async_copy(h_hbm.at[sidx.at[tsl]], rows.at[0, pl.ds(0, GTAIL)],
                          gsem.at[0]).wait()
    pltpu.sync_copy(rows.at[0, pl.ds(0, GTAIL)],
                    out_hbm.at[pl.ds(base + GFULL * GCH, GTAIL)])


def _norm_body(dout_ref, din_ref, ns_ref, nd_ref):
    dout = jnp.sum(dout_ref[...], axis=0, keepdims=True)
    din = jnp.sum(din_ref[...], axis=0, keepdims=True)
    ns_ref[...] = jnp.where(dout > 0.0, lax.rsqrt(jnp.maximum(dout, 1.0)), 0.0)
    nd_ref[...] = jnp.where(din > 0.0, lax.rsqrt(jnp.maximum(din, 1.0)), 0.0)


def _norm_kernel(doutp, dinp):
    return pl.pallas_call(
        _norm_body,
        out_shape=(
            jax.ShapeDtypeStruct((1, NPAD), jnp.float32),
            jax.ShapeDtypeStruct((1, NPAD), jnp.float32),
        ),
    )(doutp, dinp)


BM = 2000


def _mm1_body(x_ref, ns_ref, w_ref, olo_ref, ohi_ref):
    xs = x_ref[...] * ns_ref[...]
    y = jnp.dot(xs, w_ref[...], preferred_element_type=jnp.float32)
    olo_ref[...] = y[:, :DH]
    ohi_ref[...] = y[:, DH:]


def _mm1_kernel(x, ns, w):
    return pl.pallas_call(
        _mm1_body,
        grid=(N // BM,),
        in_specs=[
            pl.BlockSpec((BM, D), lambda i: (i, 0)),
            pl.BlockSpec((BM, 1), lambda i: (i, 0)),
            pl.BlockSpec((D, D), lambda i: (0, 0)),
        ],
        out_specs=[pl.BlockSpec((BM, DH), lambda i: (i, 0))] * 2,
        out_shape=[jax.ShapeDtypeStruct((N, DH), jnp.float32)] * 2,
    )(x, ns, w)


def _mid_body(alo_ref, ahi_ref, nd_ref, b1_ref, ns_ref, w_ref,
              olo_ref, ohi_ref):
    nd = nd_ref[...]
    ns = ns_ref[...]
    hlo = alo_ref[0] * nd + b1_ref[:, :DH]
    hhi = ahi_ref[0] * nd + b1_ref[:, DH:]
    hlo = jnp.where(hlo > 0.0, hlo, 0.01 * hlo) * ns
    hhi = jnp.where(hhi > 0.0, hhi, 0.01 * hhi) * ns
    y = (
        jnp.dot(hlo, w_ref[:DH, :], preferred_element_type=jnp.float32)
        + jnp.dot(hhi, w_ref[DH:, :], preferred_element_type=jnp.float32)
    )
    olo_ref[...] = y[:, :DH]
    ohi_ref[...] = y[:, DH:]


def _mid_kernel(aggp, nd, b1r, ns, w):
    return pl.pallas_call(
        _mid_body,
        grid=(N // BM,),
        in_specs=[
            pl.BlockSpec((1, BM, DH), lambda i: (0, i, 0)),
            pl.BlockSpec((1, BM, DH), lambda i: (1, i, 0)),
            pl.BlockSpec((BM, 1), lambda i: (i, 0)),
            pl.BlockSpec((1, D), lambda i: (0, 0)),
            pl.BlockSpec((BM, 1), lambda i: (i, 0)),
            pl.BlockSpec((D, D), lambda i: (0, 0)),
        ],
        out_specs=[pl.BlockSpec((BM, DH), lambda i: (i, 0))] * 2,
        out_shape=[jax.ShapeDtypeStruct((N, DH), jnp.float32)] * 2,
    )(aggp, aggp, nd, b1r, ns, w)


def _fin_body(alo_ref, ahi_ref, nd_ref, b2_ref, o_ref):
    nd = nd_ref[...]
    hlo = alo_ref[0] * nd + b2_ref[:, :DH]
    hhi = ahi_ref[0] * nd + b2_ref[:, DH:]
    o_ref[...] = jnp.concatenate([hlo, hhi], axis=-1)


def _fin_kernel(aggp, nd, b2r):
    return pl.pallas_call(
        _fin_body,
        grid=(N // BM,),
        in_specs=[
            pl.BlockSpec((1, BM, DH), lambda i: (0, i, 0)),
            pl.BlockSpec((1, BM, DH), lambda i: (1, i, 0)),
            pl.BlockSpec((BM, 1), lambda i: (i, 0)),
            pl.BlockSpec((1, D), lambda i: (0, 0)),
        ],
        out_specs=pl.BlockSpec((BM, D), lambda i: (i, 0)),
        out_shape=jax.ShapeDtypeStruct((N, D), jnp.float32),
    )(aggp, aggp, nd, b2r)


BD = 8000


def _dot_body(a_ref, b_ref, c_ref, d_ref, p_ref, n_ref):
    a = a_ref[...].astype(jnp.float32)
    b = b_ref[...].astype(jnp.float32)
    c = c_ref[...].astype(jnp.float32)
    d = d_ref[...].astype(jnp.float32)
    p_ref[...] = jnp.sum(a * b, axis=-1, keepdims=True)
    n_ref[...] = jnp.sum(c * d, axis=-1, keepdims=True)


_NBLK = E // BD


def _dot_kernel(g):
    return pl.pallas_call(
        _dot_body,
        grid=(_NBLK,),
        in_specs=[
            pl.BlockSpec((BD, D), lambda i: (i, 0)),
            pl.BlockSpec((BD, D), lambda i: (i + _NBLK, 0)),
            pl.BlockSpec((BD, D), lambda i: (i + 2 * _NBLK, 0)),
            pl.BlockSpec((BD, D), lambda i: (i + 3 * _NBLK, 0)),
        ],
        out_specs=[pl.BlockSpec((BD, 1), lambda i: (i, 0))] * 2,
        out_shape=[jax.ShapeDtypeStruct((E, 1), jnp.float32)] * 2,
    )(g, g, g, g)


@jax.jit
def kernel(edge_index, neg_edge_index, embed, W1, b1, W2, b2):
    src = edge_index[0]
    dst = edge_index[1]

    srcw = src.reshape(NW, EPW)
    dstw = dst.reshape(NW, EPW)
    srct = src.reshape(NS, ACHUNK, CH)
    dstt = dst.reshape(NS, ACHUNK, CH)

    doutp, dinp = _deg_kernel(srcw, dstw)
    nso, ndo = _norm_kernel(doutp, dinp)
    ns = nso[0, :N].reshape(N, 1)
    nd = ndo[0, :N].reshape(N, 1)

    y1lo, y1hi = _mm1_kernel(embed, ns, W1)
    aggp1 = _agg_kernel(srct, dstt, y1lo, y1hi)
    y2lo, y2hi = _mid_kernel(aggp1, nd, b1.reshape(1, D), ns, W2)
    aggp2 = _agg_kernel(srct, dstt, y2lo, y2hi)
    h = _fin_kernel(aggp2, nd, b2.reshape(1, D))

    gidx = jnp.concatenate(
        [src, dst, neg_edge_index[0], neg_edge_index[1]]
    ).reshape(NW, GPW)
    g = _gather_kernel(gidx, h)

    pos, neg = _dot_kernel(g)
    return pos, neg


# speedup vs baseline: 1.7929x; 1.0004x over previous
import functools
import jax
import jax.numpy as jnp
from jax import lax
from jax.experimental import pallas as pl
from jax.experimental.pallas import tpu as pltpu
from jax.experimental.pallas import tpu_sc as plsc

N = 10000
E = 320000
D = 128
DH = D // 2

NC, NS = 2, 16
NW = NC * NS
EPW = E // NW
EPT = E // NS
CH = 125
ACHUNK = EPT // CH
ABUF = 5
NPAD = 10240

_mesh = plsc.VectorSubcoreMesh(core_axis_name="c", subcore_axis_name="s")


@functools.partial(
    pl.kernel,
    out_type=(
        jax.ShapeDtypeStruct((NW, NPAD), jnp.float32),
        jax.ShapeDtypeStruct((NW, NPAD), jnp.float32),
    ),
    mesh=_mesh,
    scratch_types=[
        pltpu.VMEM((EPW,), jnp.int32),
        pltpu.VMEM((NPAD,), jnp.float32),
        pltpu.VMEM((NPAD,), jnp.float32),
    ],
    compiler_params=pltpu.CompilerParams(needs_layout_passes=False),
)
def _deg_kernel(src_hbm, dst_hbm, dout_hbm, din_hbm, idx_v, hist0, hist1):
    w = lax.axis_index("s") * NC + lax.axis_index("c")
    zeros = jnp.zeros((16,), jnp.float32)
    ones = jnp.ones((16,), jnp.float32)

    def zero_body(i, _):
        hist0[pl.ds(i * 16, 16)] = zeros
        hist1[pl.ds(i * 16, 16)] = zeros
        return 0

    lax.fori_loop(0, NPAD // 16, zero_body, 0)

    for ref, hist, out in ((src_hbm, hist0, dout_hbm), (dst_hbm, hist1, din_hbm)):
        pltpu.sync_copy(ref.at[w], idx_v)

        def acc_body(i, _, hist=hist):
            idx = idx_v[pl.ds(i * 16, 16)]
            plsc.addupdate_scatter(hist, [idx], ones)
            return 0

        lax.fori_loop(0, EPW // 16, acc_body, 0)
        pltpu.sync_copy(hist, out.at[w])


@functools.partial(
    pl.kernel,
    out_type=jax.ShapeDtypeStruct((NC, NPAD, DH), jnp.float32),
    mesh=_mesh,
    scratch_types=[
        pltpu.VMEM((ACHUNK, CH), jnp.int32),
        pltpu.VMEM((ACHUNK, CH), jnp.int32),
        pltpu.VMEM((ABUF, CH), jnp.int32),
        pltpu.VMEM((ABUF, CH, DH), jnp.float32),
        pltpu.VMEM_SHARED((NPAD, DH), jnp.float32),
        pltpu.SemaphoreType.DMA((ABUF,)),
    ],
    compiler_params=pltpu.CompilerParams(use_tc_tiling_on_sc=False),
)
def _agg_kernel(src_hbm, dst_hbm, y0_hbm, y1_hbm, out_hbm,
                sidx, didx, didx_s, rows, agg_sh, gsem):
    c = lax.axis_index("c")
    s = lax.axis_index("s")

    pltpu.sync_copy(src_hbm.at[s], sidx)
    pltpu.sync_copy(dst_hbm.at[s], didx)

    def stage_didx(b, j):
        for k in range(7):
            didx_s[b, pl.ds(k * 16, 16)] = didx[j, pl.ds(k * 16, 16)]
        didx_s[b, pl.ds(CH - 16, 16)] = didx[j, pl.ds(CH - 16, 16)]

    zeros = jnp.zeros((16,), jnp.float32)

    def zrow(i, _):
        for k in range(DH // 16):
            rows[0, i, pl.ds(k * 16, 16)] = zeros
        return 0

    lax.fori_loop(0, CH, zrow, 0)
    for k in range(8):
        pltpu.sync_copy(rows.at[0, pl.ds(0, 80)],
                        agg_sh.at[pl.ds(s * 640 + k * 80, 80)])
    plsc.subcore_barrier()

    def gather(j, b):
        @pl.when(c == 0)
        def _():
            pltpu.async_copy(y0_hbm.at[sidx.at[j]], rows.at[b], gsem.at[b])

        @pl.when(c == 1)
        def _():
            pltpu.async_copy(y1_hbm.at[sidx.at[j]], rows.at[b], gsem.at[b])

    def gwait(j, b):
        @pl.when(c == 0)
        def _():
            pltpu.make_async_copy(y0_hbm.at[sidx.at[j]], rows.at[b],
                                  gsem.at[b]).wait()

        @pl.when(c == 1)
        def _():
            pltpu.make_async_copy(y1_hbm.at[sidx.at[j]], rows.at[b],
                                  gsem.at[b]).wait()

    for b in range(ABUF):
        stage_didx(b, b)
        gather(b, b)

    def outer(jo, _):
        for b in range(ABUF):
            j = jo * ABUF + b
            gwait(j, b)
            pltpu.sync_copy(rows.at[b], agg_sh.at[didx_s.at[b]], add=True)

            @pl.when(jo < ACHUNK // ABUF - 1)
            def _():
                jn = j + ABUF
                stage_didx(b, jn)
                gather(jn, b)
        return 0

    lax.fori_loop(0, ACHUNK // ABUF, outer, 0)
    plsc.subcore_barrier()

    for k in range(8):
        sl = pl.ds(s * 640 + k * 80, 80)
        pltpu.sync_copy(agg_sh.at[sl], out_hbm.at[c, sl])


GPW = 4 * E // NW
GCH = 128
GFULL = GPW // GCH
GTAIL = GPW - GFULL * GCH
GBUF = 4


@functools.partial(
    pl.kernel,
    out_type=jax.ShapeDtypeStruct((4 * E, D), jnp.float32),
    mesh=_mesh,
    scratch_types=[
        pltpu.VMEM((GPW,), jnp.int32),
        pltpu.VMEM((GBUF, GCH, D), jnp.float32),
        pltpu.SemaphoreType.DMA((GBUF,)),
    ],
)
def _gather_kernel(gidx_hbm, h_hbm, out_hbm, sidx, rows, gsem):
    w = lax.axis_index("s") * NC + lax.axis_index("c")
    base = w * GPW
    pltpu.sync_copy(gidx_hbm.at[w], sidx)

    for b in range(GBUF):
        pltpu.async_copy(h_hbm.at[sidx.at[pl.ds(b * GCH, GCH)]], rows.at[b],
                         gsem.at[b])

    def outer(jo, _):
        for b in range(GBUF):
            j = jo * GBUF + b
            pltpu.make_async_copy(h_hbm.at[sidx.at[pl.ds(j * GCH, GCH)]],
                                  rows.at[b], gsem.at[b]).wait()
            pltpu.sync_copy(rows.at[b], out_hbm.at[pl.ds(base + j * GCH, GCH)])

            @pl.when(jo < GFULL // GBUF - 1)
            def _():
                jn = j + GBUF
                pltpu.async_copy(h_hbm.at[sidx.at[pl.ds(jn * GCH, GCH)]],
                                 rows.at[b], gsem.at[b])
        return 0

    lax.fori_loop(0, GFULL // GBUF, outer, 0)

    tsl = pl.ds(GFULL * GCH, GTAIL)
    pltpu.async_copy(h_hbm.at[sidx.at[tsl]], rows.at[0, pl.ds(0, GTAIL)],
                     gsem.at[0])
    pltpu.make_async_copy(h_hbm.at[sidx.at[tsl]], rows.at[0, pl.ds(0, GTAIL)],
                          gsem.at[0]).wait()
    pltpu.sync_copy(rows.at[0, pl.ds(0, GTAIL)],
                    out_hbm.at[pl.ds(base + GFULL * GCH, GTAIL)])


def _norm_body(dout_ref, din_ref, ns_ref, nd_ref):
    dout = jnp.sum(dout_ref[...], axis=0, keepdims=True)
    din = jnp.sum(din_ref[...], axis=0, keepdims=True)
    ns_ref[...] = jnp.where(dout > 0.0, lax.rsqrt(jnp.maximum(dout, 1.0)), 0.0)
    nd_ref[...] = jnp.where(din > 0.0, lax.rsqrt(jnp.maximum(din, 1.0)), 0.0)


def _norm_kernel(doutp, dinp):
    return pl.pallas_call(
        _norm_body,
        out_shape=(
            jax.ShapeDtypeStruct((1, NPAD), jnp.float32),
            jax.ShapeDtypeStruct((1, NPAD), jnp.float32),
        ),
    )(doutp, dinp)


BM = 2000


def _mm1_body(x_ref, ns_ref, w_ref, olo_ref, ohi_ref):
    xs = x_ref[...] * ns_ref[...]
    y = jnp.dot(xs, w_ref[...], preferred_element_type=jnp.float32)
    olo_ref[...] = y[:, :DH]
    ohi_ref[...] = y[:, DH:]


def _mm1_kernel(x, ns, w):
    return pl.pallas_call(
        _mm1_body,
        grid=(N // BM,),
        in_specs=[
            pl.BlockSpec((BM, D), lambda i: (i, 0)),
            pl.BlockSpec((BM, 1), lambda i: (i, 0)),
            pl.BlockSpec((D, D), lambda i: (0, 0)),
        ],
        out_specs=[pl.BlockSpec((BM, DH), lambda i: (i, 0))] * 2,
        out_shape=[jax.ShapeDtypeStruct((N, DH), jnp.float32)] * 2,
    )(x, ns, w)


def _mid_body(alo_ref, ahi_ref, nd_ref, b1_ref, ns_ref, w_ref,
              olo_ref, ohi_ref):
    nd = nd_ref[...]
    ns = ns_ref[...]
    hlo = alo_ref[0] * nd + b1_ref[:, :DH]
    hhi = ahi_ref[0] * nd + b1_ref[:, DH:]
    hlo = jnp.where(hlo > 0.0, hlo, 0.01 * hlo) * ns
    hhi = jnp.where(hhi > 0.0, hhi, 0.01 * hhi) * ns
    y = (
        jnp.dot(hlo, w_ref[:DH, :], preferred_element_type=jnp.float32)
        + jnp.dot(hhi, w_ref[DH:, :], preferred_element_type=jnp.float32)
    )
    olo_ref[...] = y[:, :DH]
    ohi_ref[...] = y[:, DH:]


def _mid_kernel(aggp, nd, b1r, ns, w):
    return pl.pallas_call(
        _mid_body,
        grid=(N // BM,),
        in_specs=[
            pl.BlockSpec((1, BM, DH), lambda i: (0, i, 0)),
            pl.BlockSpec((1, BM, DH), lambda i: (1, i, 0)),
            pl.BlockSpec((BM, 1), lambda i: (i, 0)),
            pl.BlockSpec((1, D), lambda i: (0, 0)),
            pl.BlockSpec((BM, 1), lambda i: (i, 0)),
            pl.BlockSpec((D, D), lambda i: (0, 0)),
        ],
        out_specs=[pl.BlockSpec((BM, DH), lambda i: (i, 0))] * 2,
        out_shape=[jax.ShapeDtypeStruct((N, DH), jnp.float32)] * 2,
    )(aggp, aggp, nd, b1r, ns, w)


def _fin_body(alo_ref, ahi_ref, nd_ref, b2_ref, o_ref):
    nd = nd_ref[...]
    hlo = alo_ref[0] * nd + b2_ref[:, :DH]
    hhi = ahi_ref[0] * nd + b2_ref[:, DH:]
    o_ref[...] = jnp.concatenate([hlo, hhi], axis=-1)


def _fin_kernel(aggp, nd, b2r):
    return pl.pallas_call(
        _fin_body,
        grid=(N // BM,),
        in_specs=[
            pl.BlockSpec((1, BM, DH), lambda i: (0, i, 0)),
            pl.BlockSpec((1, BM, DH), lambda i: (1, i, 0)),
            pl.BlockSpec((BM, 1), lambda i: (i, 0)),
            pl.BlockSpec((1, D), lambda i: (0, 0)),
        ],
        out_specs=pl.BlockSpec((BM, D), lambda i: (i, 0)),
        out_shape=jax.ShapeDtypeStruct((N, D), jnp.float32),
    )(aggp, aggp, nd, b2r)


BD = 8000


def _dot_body(a_ref, b_ref, c_ref, d_ref, p_ref, n_ref):
    a = a_ref[...].astype(jnp.float32)
    b = b_ref[...].astype(jnp.float32)
    c = c_ref[...].astype(jnp.float32)
    d = d_ref[...].astype(jnp.float32)
    p_ref[...] = jnp.sum(a * b, axis=-1, keepdims=True)
    n_ref[...] = jnp.sum(c * d, axis=-1, keepdims=True)


_NBLK = E // BD


def _dot_kernel(g):
    return pl.pallas_call(
        _dot_body,
        grid=(_NBLK,),
        in_specs=[
            pl.BlockSpec((BD, D), lambda i: (i, 0)),
            pl.BlockSpec((BD, D), lambda i: (i + _NBLK, 0)),
            pl.BlockSpec((BD, D), lambda i: (i + 2 * _NBLK, 0)),
            pl.BlockSpec((BD, D), lambda i: (i + 3 * _NBLK, 0)),
        ],
        out_specs=[pl.BlockSpec((BD, 1), lambda i: (i, 0))] * 2,
        out_shape=[jax.ShapeDtypeStruct((E, 1), jnp.float32)] * 2,
    )(g, g, g, g)


@jax.jit
def kernel(edge_index, neg_edge_index, embed, W1, b1, W2, b2):
    src = edge_index[0]
    dst = edge_index[1]

    srcw = src.reshape(NW, EPW)
    dstw = dst.reshape(NW, EPW)
    srct = src.reshape(NS, ACHUNK, CH)
    dstt = dst.reshape(NS, ACHUNK, CH)

    doutp, dinp = _deg_kernel(srcw, dstw)
    nso, ndo = _norm_kernel(doutp, dinp)
    ns = nso[0, :N].reshape(N, 1)
    nd = ndo[0, :N].reshape(N, 1)

    y1lo, y1hi = _mm1_kernel(embed, ns, W1)
    aggp1 = _agg_kernel(srct, dstt, y1lo, y1hi)
    y2lo, y2hi = _mid_kernel(aggp1, nd, b1.reshape(1, D), ns, W2)
    aggp2 = _agg_kernel(srct, dstt, y2lo, y2hi)
    h = _fin_kernel(aggp2, nd, b2.reshape(1, D))

    gidx = jnp.concatenate(
        [src, dst, neg_edge_index[0], neg_edge_index[1]]
    ).reshape(NW, GPW)
    g = _gather_kernel(gidx, h)

    pos, neg = _dot_kernel(g)
    return pos, neg
